# baseline (speedup 1.0000x reference)
import functools

import jax
import jax.numpy as jnp
from jax import lax
from jax.experimental import pallas as pl
from jax.experimental.pallas import tpu as pltpu
from jax.experimental.pallas import tpu_sc as plsc

N = 10000
DIN = 128
DH = 64
E = 320000
RPT = 640
N_PAD = RPT * 16
SR = 160
CH = 128
NCH = 2560
E_PAD = NCH * CH
CPT_DUAL = NCH // 16
CPT_HOP = NCH // 32
SEG = 80
DEGW = 16
BR = 1000
NB = 4

_MESH = plsc.VectorSubcoreMesh(core_axis_name="c", subcore_axis_name="s")
_SC_PARAMS = pltpu.CompilerParams(use_tc_tiling_on_sc=False,
                                  needs_layout_passes=False)

_Z16 = functools.partial(jnp.zeros, (16,), jnp.float32)


def _zero_rows(buf, nrows, ncols):
    def body(i, _):
        for q in range(ncols // 16):
            buf[i, pl.ds(q * 16, 16)] = _Z16()
        return 0
    lax.fori_loop(0, nrows, body, 0)


def _edge_loop(tq_hbm, acc_sh, src_v, dst_v, rows_i, rows_f,
               gsems, ssems, nchunks):
    def g_desc(j, b):
        return pltpu.make_async_copy(tq_hbm.at[src_v.at[j]], rows_i[b],
                                     gsems[b])

    def s_desc(j, b):
        return pltpu.make_async_copy(rows_f[b], acc_sh.at[dst_v.at[j]],
                                     ssems[b])

    for b in range(NB - 1):
        g_desc(b, b).start()

    def group(g, _):
        for b in range(NB):
            j = g * NB + b

            @pl.when(j >= 1)
            def _():
                s_desc(j - 1, (b - 1) % NB).wait()

            @pl.when(j + NB - 1 < nchunks)
            def _():
                g_desc(j + NB - 1, (b + NB - 1) % NB).start()

            g_desc(j, b).wait()

            @plsc.parallel_loop(0, CH, unroll=16)
            def conv(r, b=b):
                for gg in range(DH // 32):
                    w = rows_i[b][r, pl.ds(gg * 16, 16)]
                    bf = plsc.bitcast(w, jnp.bfloat16)
                    a0, a1 = plsc.unpack(bf, format=plsc.PackFormat.INTERLEAVED)
                    rows_f[b][r, pl.ds(gg * 32, 16)] = a0
                    rows_f[b][r, pl.ds(gg * 32 + 16, 16)] = a1

            s_desc(j, b).start(add=True)
        return 0
    lax.fori_loop(0, nchunks // NB, group, 0)
    s_desc(nchunks - 1, (nchunks - 1) % NB).wait()


def _hist_body(dsts_hbm, out_hbm, deg_sh, dst_v, ones_v, stage_v,
               sem0, sem1, sem2, sem3):
    c = lax.axis_index("c")
    s = lax.axis_index("s")
    one16 = jnp.full((16,), 1.0, jnp.float32)
    ssems = [sem0, sem1, sem2, sem3]

    for i in range(CH // 16):
        ones_v[pl.ds(i * 16, 16)] = one16
    def zrow(i, _):
        stage_v[pl.ds(i * 16, 16)] = _Z16()
        return 0
    lax.fori_loop(0, SR // 16, zrow, 0)
    for k in range(RPT // SR):
        pltpu.sync_copy(stage_v, deg_sh.at[pl.ds(s * RPT + k * SR, SR)])
    plsc.subcore_barrier()

    pltpu.sync_copy(dsts_hbm.at[c, pl.ds(s * CPT_DUAL, CPT_DUAL)], dst_v)

    def s_desc(j, b):
        return pltpu.make_async_copy(ones_v, deg_sh.at[dst_v.at[j]], ssems[b])

    def group(g, _):
        for b in range(4):
            j = g * 4 + b

            @pl.when(j >= 4)
            def _():
                s_desc(j - 4, b).wait()

            s_desc(j, b).start(add=True)
        return 0
    lax.fori_loop(0, CPT_DUAL // 4, group, 0)
    for b in range(4):
        s_desc(CPT_DUAL - 4 + b, b).wait()

    plsc.subcore_barrier()
    for k in range(RPT // SR):
        pltpu.sync_copy(deg_sh.at[pl.ds(s * RPT + k * SR, SR)], stage_v)
        pltpu.sync_copy(stage_v, out_hbm.at[c, pl.ds(s * RPT + k * SR, SR)])


_hist = pl.kernel(
    _hist_body,
    out_type=jax.ShapeDtypeStruct((2, N_PAD), jnp.float32),
    mesh=_MESH,
    compiler_params=_SC_PARAMS,
    scratch_types=[
        pltpu.VMEM_SHARED((N_PAD,), jnp.float32),
        pltpu.VMEM((CPT_DUAL, CH), jnp.int32),
        pltpu.VMEM((CH,), jnp.float32),
        pltpu.VMEM((SR,), jnp.float32),
    ] + [pltpu.SemaphoreType.DMA] * 4,
)


def _dual_body(t1q_hbm, t2q_hbm, srcs_hbm, dsts_hbm, out_hbm,
               acc_sh, src_v, dst_v,
               ri0, ri1, ri2, ri3, rf0, rf1, rf2, rf3, stage_v,
               g0, g1, g2, g3, s0, s1, s2, s3):
    c = lax.axis_index("c")
    s = lax.axis_index("s")
    rows_i = [ri0, ri1, ri2, ri3]
    rows_f = [rf0, rf1, rf2, rf3]
    gsems = [g0, g1, g2, g3]
    ssems = [s0, s1, s2, s3]
    _zero_rows(stage_v, SR, DH)
    for k in range(RPT // SR):
        pltpu.sync_copy(stage_v, acc_sh.at[pl.ds(s * RPT + k * SR, SR)])
    plsc.subcore_barrier()

    for seg in range(CPT_DUAL // SEG):
        base = s * CPT_DUAL + seg * SEG
        pltpu.sync_copy(srcs_hbm.at[c, pl.ds(base, SEG)], src_v)
        pltpu.sync_copy(dsts_hbm.at[c, pl.ds(base, SEG)], dst_v)

        @pl.when(c == 0)
        def _():
            _edge_loop(t1q_hbm, acc_sh, src_v, dst_v, rows_i, rows_f,
                       gsems, ssems, SEG)

        @pl.when(c == 1)
        def _():
            _edge_loop(t2q_hbm, acc_sh, src_v, dst_v, rows_i, rows_f,
                       gsems, ssems, SEG)

    plsc.subcore_barrier()
    for k in range(RPT // SR):
        pltpu.sync_copy(acc_sh.at[pl.ds(s * RPT + k * SR, SR)], stage_v)
        pltpu.sync_copy(stage_v, out_hbm.at[c, pl.ds(s * RPT + k * SR, SR)])


_dual = pl.kernel(
    _dual_body,
    out_type=jax.ShapeDtypeStruct((2, N_PAD, DH), jnp.float32),
    mesh=_MESH,
    compiler_params=_SC_PARAMS,
    scratch_types=[
        pltpu.VMEM_SHARED((N_PAD, DH), jnp.float32),
        pltpu.VMEM((SEG, CH), jnp.int32),
        pltpu.VMEM((SEG, CH), jnp.int32),
    ] + [pltpu.VMEM((CH, DH // 2), jnp.uint32)] * NB
      + [pltpu.VMEM((CH, DH), jnp.float32)] * NB + [
        pltpu.VMEM((SR, DH), jnp.float32),
    ] + [pltpu.SemaphoreType.DMA] * (2 * NB),
)


SEGH = 40


def _hop2_body(accs_hbm, t2_hbm, deg_hbm, srcs_hbm, dsts_hbm,
               out_hbm, tq_hbm,
               acc_sh, src_v, dst_v,
               ri0, ri1, ri2, ri3, rf0, rf1, rf2, rf3, stage_v, b_v, d_v, q_v,
               g0, g1, g2, g3, s0, s1, s2, s3):
    c = lax.axis_index("c")
    s = lax.axis_index("s")
    wid = s * 2 + c
    rows_i = [ri0, ri1, ri2, ri3]
    rows_f = [rf0, rf1, rf2, rf3]
    gsems = [g0, g1, g2, g3]
    ssems = [s0, s1, s2, s3]
    _zero_rows(stage_v, SR, DH)
    for k in range(RPT // SR):
        pltpu.sync_copy(stage_v, acc_sh.at[pl.ds(s * RPT + k * SR, SR)])

    for k in range(RPT // SR):
        rows = s * RPT + k * SR
        pltpu.sync_copy(accs_hbm.at[1, pl.ds(rows, SR)], stage_v)
        pltpu.sync_copy(t2_hbm.at[pl.ds(rows, SR)], b_v)
        pltpu.sync_copy(deg_hbm.at[1, pl.ds(rows, SR)], d_v)

        @plsc.parallel_loop(0, SR // 16)
        def t2p_grp(g16):
            r0 = g16 * 16
            dvec = d_v[pl.ds(r0, 16)]
            for i in range(16):
                r = r0 + i
                rv = 1.0 / (jnp.full((16,), dvec[i], jnp.float32) + 1.0)
                for gg in range(DH // 32):
                    lo = (stage_v[r, pl.ds(gg * 32, 16)]
                          + b_v[r, pl.ds(gg * 32, 16)]) * rv
                    hi = (stage_v[r, pl.ds(gg * 32 + 16, 16)]
                          + b_v[r, pl.ds(gg * 32 + 16, 16)]) * rv
                    w = plsc.pack(lo, hi, format=plsc.PackFormat.INTERLEAVED)
                    q_v[r, pl.ds(gg * 16, 16)] = plsc.bitcast(w, jnp.uint32)

        pltpu.sync_copy(q_v, tq_hbm.at[c, pl.ds(rows, SR)])
    plsc.subcore_barrier()

    for seg in range(CPT_HOP // SEGH):
        base = wid * CPT_HOP + seg * SEGH
        pltpu.sync_copy(srcs_hbm.at[1, pl.ds(base, SEGH)], src_v)
        pltpu.sync_copy(dsts_hbm.at[1, pl.ds(base, SEGH)], dst_v)

        @pl.when(c == 0)
        def _():
            _edge_loop(tq_hbm.at[0], acc_sh, src_v, dst_v, rows_i, rows_f,
                       gsems, ssems, SEGH)

        @pl.when(c == 1)
        def _():
            _edge_loop(tq_hbm.at[1], acc_sh, src_v, dst_v, rows_i, rows_f,
                       gsems, ssems, SEGH)

    plsc.subcore_barrier()
    for k in range(RPT // SR):
        pltpu.sync_copy(acc_sh.at[pl.ds(s * RPT + k * SR, SR)], stage_v)
        pltpu.sync_copy(stage_v, out_hbm.at[c, pl.ds(s * RPT + k * SR, SR)])


_hop2 = pl.kernel(
    _hop2_body,
    out_type=[jax.ShapeDtypeStruct((2, N_PAD, DH), jnp.float32),
              jax.ShapeDtypeStruct((2, N_PAD, DH // 2), jnp.uint32)],
    mesh=_MESH,
    compiler_params=_SC_PARAMS,
    scratch_types=[
        pltpu.VMEM_SHARED((N_PAD, DH), jnp.float32),
        pltpu.VMEM((SEGH, CH), jnp.int32),
        pltpu.VMEM((SEGH, CH), jnp.int32),
    ] + [pltpu.VMEM((CH, DH // 2), jnp.uint32)] * NB
      + [pltpu.VMEM((CH, DH), jnp.float32)] * NB + [
        pltpu.VMEM((SR, DH), jnp.float32),
        pltpu.VMEM((SR, DH), jnp.float32),
        pltpu.VMEM((SR,), jnp.float32),
        pltpu.VMEM((SR, DH // 2), jnp.uint32),
    ] + [pltpu.SemaphoreType.DMA] * (2 * NB),
)


def _pack_tbl(t):
    lo = jnp.concatenate([t[:, 0:16], t[:, 32:48]], axis=1)
    hi = jnp.concatenate([t[:, 16:32], t[:, 48:64]], axis=1)
    lb = lax.bitcast_convert_type(lo.astype(jnp.bfloat16), jnp.uint16)
    hb = lax.bitcast_convert_type(hi.astype(jnp.bfloat16), jnp.uint16)
    return lb.astype(jnp.uint32) | (hb.astype(jnp.uint32) << 16)


def _pre_body(x_ref, w_ref, deg0_ref, deg1_ref,
              t1_ref, t2_ref, t1q_ref, t2q_ref):
    h = jnp.dot(x_ref[...], w_ref[...], preferred_element_type=jnp.float32)
    d0 = lax.rsqrt(deg0_ref[...] + 1.0)
    d1 = lax.rsqrt(deg1_ref[...] + 1.0)
    t1 = h[:, :DH] * d0
    t2 = h[:, DH:] * d1
    t1_ref[...] = t1
    t2_ref[...] = t2
    t1q_ref[...] = _pack_tbl(t1)
    t2q_ref[...] = _pack_tbl(t2)


_pre = pl.pallas_call(
    _pre_body,
    grid=(N // BR,),
    in_specs=[
        pl.BlockSpec((BR, DIN), lambda i: (i, 0)),
        pl.BlockSpec((DIN, DIN), lambda i: (0, 0)),
        pl.BlockSpec((BR, 1), lambda i: (i, 0)),
        pl.BlockSpec((BR, 1), lambda i: (i, 0)),
    ],
    out_specs=[pl.BlockSpec((BR, DH), lambda i: (i, 0))] * 2
              + [pl.BlockSpec((BR, DH // 2), lambda i: (i, 0))] * 2,
    out_shape=[jax.ShapeDtypeStruct((N_PAD, DH), jnp.float32)] * 2
             + [jax.ShapeDtypeStruct((N, DH // 2), jnp.uint32)] * 2,
)


def _fin_body(accs_ref, acch_ref, t1_ref, t2_ref, deg0_ref, deg1_ref,
              b1_ref, b2_ref, o_ref):
    d0 = lax.rsqrt(deg0_ref[...] + 1.0)
    d1 = lax.rsqrt(deg1_ref[...] + 1.0)
    left = (accs_ref[0] + t1_ref[...]) * d0 + b1_ref[...]
    t2p = (accs_ref[1] + t2_ref[...]) / (deg1_ref[...] + 1.0)
    right = (acch_ref[0] + acch_ref[1] + t2p) * d1 + b2_ref[...]
    o_ref[...] = jnp.concatenate([left, right], axis=1)


_fin = pl.pallas_call(
    _fin_body,
    grid=(N // BR,),
    in_specs=[
        pl.BlockSpec((2, BR, DH), lambda i: (0, i, 0)),
        pl.BlockSpec((2, BR, DH), lambda i: (0, i, 0)),
        pl.BlockSpec((BR, DH), lambda i: (i, 0)),
        pl.BlockSpec((BR, DH), lambda i: (i, 0)),
        pl.BlockSpec((BR, 1), lambda i: (i, 0)),
        pl.BlockSpec((BR, 1), lambda i: (i, 0)),
        pl.BlockSpec((1, DH), lambda i: (0, 0)),
        pl.BlockSpec((1, DH), lambda i: (0, 0)),
    ],
    out_specs=pl.BlockSpec((BR, DIN), lambda i: (i, 0)),
    out_shape=jax.ShapeDtypeStruct((N, DIN), jnp.float32),
)


def kernel(x, edges, W1, b1, W2, b2):
    pad = E_PAD - E
    zpad = jnp.zeros((pad,), jnp.int32)
    dpad = jnp.full((pad,), N, jnp.int32)
    srcs = jnp.stack([
        jnp.concatenate([edges[0, 0], zpad]),
        jnp.concatenate([edges[1, 0], zpad]),
    ]).reshape(2, NCH, CH)
    dsts = jnp.stack([
        jnp.concatenate([edges[0, 1], dpad]),
        jnp.concatenate([edges[1, 1], dpad]),
    ]).reshape(2, NCH, CH)

    degp = _hist(dsts)
    deg0 = degp[0, :N].reshape(N, 1)
    deg1 = degp[1, :N].reshape(N, 1)

    wcat = jnp.concatenate([W1, W2], axis=1)
    t1, t2, t1q, t2q = _pre(x, wcat, deg0, deg1)
    accs = _dual(t1q, t2q, srcs, dsts)
    acch, _tq = _hop2(accs, t2, degp, srcs, dsts)
    return _fin(accs, acch, t1, t2, deg0, deg1,
                b1.reshape(1, DH), b2.reshape(1, DH))


# speedup vs baseline: 1.0039x; 1.0039x over previous
import functools

import jax
import jax.numpy as jnp
from jax import lax
from jax.experimental import pallas as pl
from jax.experimental.pallas import tpu as pltpu
from jax.experimental.pallas import tpu_sc as plsc

N = 10000
DIN = 128
DH = 64
E = 320000
RPT = 640
N_PAD = RPT * 16
SR = 160
CH = 128
NCH = 2560
E_PAD = NCH * CH
CPT_DUAL = NCH // 16
CPT_HOP = NCH // 32
SEG = 80
DEGW = 16
BR = 1000
NB = 4

_MESH = plsc.VectorSubcoreMesh(core_axis_name="c", subcore_axis_name="s")
_SC_PARAMS = pltpu.CompilerParams(use_tc_tiling_on_sc=False,
                                  needs_layout_passes=False)

_Z16 = functools.partial(jnp.zeros, (16,), jnp.float32)


def _zero_rows(buf, nrows, ncols):
    def body(i, _):
        for q in range(ncols // 16):
            buf[i, pl.ds(q * 16, 16)] = _Z16()
        return 0
    lax.fori_loop(0, nrows, body, 0)


def _edge_loop(tq_hbm, acc_sh, src_v, dst_v, rows_i, rows_f,
               gsems, ssems, nchunks):
    def g_desc(j, b):
        return pltpu.make_async_copy(tq_hbm.at[src_v.at[j]], rows_i[b],
                                     gsems[b])

    def s_desc(j, b):
        return pltpu.make_async_copy(rows_f[b], acc_sh.at[dst_v.at[j]],
                                     ssems[b])

    for b in range(NB - 1):
        g_desc(b, b).start()

    def group(g, _):
        for b in range(NB):
            j = g * NB + b

            @pl.when(j >= 1)
            def _():
                s_desc(j - 1, (b - 1) % NB).wait()

            @pl.when(j + NB - 1 < nchunks)
            def _():
                g_desc(j + NB - 1, (b + NB - 1) % NB).start()

            g_desc(j, b).wait()

            @plsc.parallel_loop(0, CH, unroll=8)
            def conv(r, b=b):
                for gg in range(DH // 32):
                    w = rows_i[b][r, pl.ds(gg * 16, 16)]
                    bf = plsc.bitcast(w, jnp.bfloat16)
                    a0, a1 = plsc.unpack(bf, format=plsc.PackFormat.INTERLEAVED)
                    rows_f[b][r, pl.ds(gg * 32, 16)] = a0
                    rows_f[b][r, pl.ds(gg * 32 + 16, 16)] = a1

            s_desc(j, b).start(add=True)
        return 0
    lax.fori_loop(0, nchunks // NB, group, 0)
    s_desc(nchunks - 1, (nchunks - 1) % NB).wait()


def _hist_body(dsts_hbm, out_hbm, deg_sh, dst_v, ones_v, stage_v,
               sem0, sem1, sem2, sem3):
    c = lax.axis_index("c")
    s = lax.axis_index("s")
    one16 = jnp.full((16,), 1.0, jnp.float32)
    ssems = [sem0, sem1, sem2, sem3]

    for i in range(CH // 16):
        ones_v[pl.ds(i * 16, 16)] = one16
    def zrow(i, _):
        stage_v[pl.ds(i * 16, 16)] = _Z16()
        return 0
    lax.fori_loop(0, SR // 16, zrow, 0)
    for k in range(RPT // SR):
        pltpu.sync_copy(stage_v, deg_sh.at[pl.ds(s * RPT + k * SR, SR)])
    plsc.subcore_barrier()

    pltpu.sync_copy(dsts_hbm.at[c, pl.ds(s * CPT_DUAL, CPT_DUAL)], dst_v)

    def s_desc(j, b):
        return pltpu.make_async_copy(ones_v, deg_sh.at[dst_v.at[j]], ssems[b])

    def group(g, _):
        for b in range(4):
            j = g * 4 + b

            @pl.when(j >= 4)
            def _():
                s_desc(j - 4, b).wait()

            s_desc(j, b).start(add=True)
        return 0
    lax.fori_loop(0, CPT_DUAL // 4, group, 0)
    for b in range(4):
        s_desc(CPT_DUAL - 4 + b, b).wait()

    plsc.subcore_barrier()
    for k in range(RPT // SR):
        pltpu.sync_copy(deg_sh.at[pl.ds(s * RPT + k * SR, SR)], stage_v)
        pltpu.sync_copy(stage_v, out_hbm.at[c, pl.ds(s * RPT + k * SR, SR)])


_hist = pl.kernel(
    _hist_body,
    out_type=jax.ShapeDtypeStruct((2, N_PAD), jnp.float32),
    mesh=_MESH,
    compiler_params=_SC_PARAMS,
    scratch_types=[
        pltpu.VMEM_SHARED((N_PAD,), jnp.float32),
        pltpu.VMEM((CPT_DUAL, CH), jnp.int32),
        pltpu.VMEM((CH,), jnp.float32),
        pltpu.VMEM((SR,), jnp.float32),
    ] + [pltpu.SemaphoreType.DMA] * 4,
)


def _dual_body(t1q_hbm, t2q_hbm, srcs_hbm, dsts_hbm, out_hbm,
               acc_sh, src_v, dst_v,
               ri0, ri1, ri2, ri3, rf0, rf1, rf2, rf3, stage_v,
               g0, g1, g2, g3, s0, s1, s2, s3):
    c = lax.axis_index("c")
    s = lax.axis_index("s")
    rows_i = [ri0, ri1, ri2, ri3]
    rows_f = [rf0, rf1, rf2, rf3]
    gsems = [g0, g1, g2, g3]
    ssems = [s0, s1, s2, s3]
    _zero_rows(stage_v, SR, DH)
    for k in range(RPT // SR):
        pltpu.sync_copy(stage_v, acc_sh.at[pl.ds(s * RPT + k * SR, SR)])
    plsc.subcore_barrier()

    for seg in range(CPT_DUAL // SEG):
        base = s * CPT_DUAL + seg * SEG
        pltpu.sync_copy(srcs_hbm.at[c, pl.ds(base, SEG)], src_v)
        pltpu.sync_copy(dsts_hbm.at[c, pl.ds(base, SEG)], dst_v)

        @pl.when(c == 0)
        def _():
            _edge_loop(t1q_hbm, acc_sh, src_v, dst_v, rows_i, rows_f,
                       gsems, ssems, SEG)

        @pl.when(c == 1)
        def _():
            _edge_loop(t2q_hbm, acc_sh, src_v, dst_v, rows_i, rows_f,
                       gsems, ssems, SEG)

    plsc.subcore_barrier()
    for k in range(RPT // SR):
        pltpu.sync_copy(acc_sh.at[pl.ds(s * RPT + k * SR, SR)], stage_v)
        pltpu.sync_copy(stage_v, out_hbm.at[c, pl.ds(s * RPT + k * SR, SR)])


_dual = pl.kernel(
    _dual_body,
    out_type=jax.ShapeDtypeStruct((2, N_PAD, DH), jnp.float32),
    mesh=_MESH,
    compiler_params=_SC_PARAMS,
    scratch_types=[
        pltpu.VMEM_SHARED((N_PAD, DH), jnp.float32),
        pltpu.VMEM((SEG, CH), jnp.int32),
        pltpu.VMEM((SEG, CH), jnp.int32),
    ] + [pltpu.VMEM((CH, DH // 2), jnp.uint32)] * NB
      + [pltpu.VMEM((CH, DH), jnp.float32)] * NB + [
        pltpu.VMEM((SR, DH), jnp.float32),
    ] + [pltpu.SemaphoreType.DMA] * (2 * NB),
)


SEGH = 40


def _hop2_body(accs_hbm, t2_hbm, deg_hbm, srcs_hbm, dsts_hbm,
               out_hbm, tq_hbm,
               acc_sh, src_v, dst_v,
               ri0, ri1, ri2, ri3, rf0, rf1, rf2, rf3, stage_v, b_v, d_v, q_v,
               g0, g1, g2, g3, s0, s1, s2, s3):
    c = lax.axis_index("c")
    s = lax.axis_index("s")
    wid = s * 2 + c
    rows_i = [ri0, ri1, ri2, ri3]
    rows_f = [rf0, rf1, rf2, rf3]
    gsems = [g0, g1, g2, g3]
    ssems = [s0, s1, s2, s3]
    _zero_rows(stage_v, SR, DH)
    for k in range(RPT // SR):
        pltpu.sync_copy(stage_v, acc_sh.at[pl.ds(s * RPT + k * SR, SR)])

    for k in range(RPT // SR):
        rows = s * RPT + k * SR
        pltpu.sync_copy(accs_hbm.at[1, pl.ds(rows, SR)], stage_v)
        pltpu.sync_copy(t2_hbm.at[pl.ds(rows, SR)], b_v)
        pltpu.sync_copy(deg_hbm.at[1, pl.ds(rows, SR)], d_v)

        @plsc.parallel_loop(0, SR // 16)
        def t2p_grp(g16):
            r0 = g16 * 16
            dvec = d_v[pl.ds(r0, 16)]
            for i in range(16):
                r = r0 + i
                rv = 1.0 / (jnp.full((16,), dvec[i], jnp.float32) + 1.0)
                for gg in range(DH // 32):
                    lo = (stage_v[r, pl.ds(gg * 32, 16)]
                          + b_v[r, pl.ds(gg * 32, 16)]) * rv
                    hi = (stage_v[r, pl.ds(gg * 32 + 16, 16)]
                          + b_v[r, pl.ds(gg * 32 + 16, 16)]) * rv
                    w = plsc.pack(lo, hi, format=plsc.PackFormat.INTERLEAVED)
                    q_v[r, pl.ds(gg * 16, 16)] = plsc.bitcast(w, jnp.uint32)

        pltpu.sync_copy(q_v, tq_hbm.at[c, pl.ds(rows, SR)])
    plsc.subcore_barrier()

    for seg in range(CPT_HOP // SEGH):
        base = wid * CPT_HOP + seg * SEGH
        pltpu.sync_copy(srcs_hbm.at[1, pl.ds(base, SEGH)], src_v)
        pltpu.sync_copy(dsts_hbm.at[1, pl.ds(base, SEGH)], dst_v)

        @pl.when(c == 0)
        def _():
            _edge_loop(tq_hbm.at[0], acc_sh, src_v, dst_v, rows_i, rows_f,
                       gsems, ssems, SEGH)

        @pl.when(c == 1)
        def _():
            _edge_loop(tq_hbm.at[1], acc_sh, src_v, dst_v, rows_i, rows_f,
                       gsems, ssems, SEGH)

    plsc.subcore_barrier()
    for k in range(RPT // SR):
        pltpu.sync_copy(acc_sh.at[pl.ds(s * RPT + k * SR, SR)], stage_v)
        pltpu.sync_copy(stage_v, out_hbm.at[c, pl.ds(s * RPT + k * SR, SR)])


_hop2 = pl.kernel(
    _hop2_body,
    out_type=[jax.ShapeDtypeStruct((2, N_PAD, DH), jnp.float32),
              jax.ShapeDtypeStruct((2, N_PAD, DH // 2), jnp.uint32)],
    mesh=_MESH,
    compiler_params=_SC_PARAMS,
    scratch_types=[
        pltpu.VMEM_SHARED((N_PAD, DH), jnp.float32),
        pltpu.VMEM((SEGH, CH), jnp.int32),
        pltpu.VMEM((SEGH, CH), jnp.int32),
    ] + [pltpu.VMEM((CH, DH // 2), jnp.uint32)] * NB
      + [pltpu.VMEM((CH, DH), jnp.float32)] * NB + [
        pltpu.VMEM((SR, DH), jnp.float32),
        pltpu.VMEM((SR, DH), jnp.float32),
        pltpu.VMEM((SR,), jnp.float32),
        pltpu.VMEM((SR, DH // 2), jnp.uint32),
    ] + [pltpu.SemaphoreType.DMA] * (2 * NB),
)


def _pack_tbl(t):
    lo = jnp.concatenate([t[:, 0:16], t[:, 32:48]], axis=1)
    hi = jnp.concatenate([t[:, 16:32], t[:, 48:64]], axis=1)
    lb = lax.bitcast_convert_type(lo.astype(jnp.bfloat16), jnp.uint16)
    hb = lax.bitcast_convert_type(hi.astype(jnp.bfloat16), jnp.uint16)
    return lb.astype(jnp.uint32) | (hb.astype(jnp.uint32) << 16)


def _pre_body(x_ref, w_ref, deg0_ref, deg1_ref,
              t1_ref, t2_ref, t1q_ref, t2q_ref):
    h = jnp.dot(x_ref[...], w_ref[...], preferred_element_type=jnp.float32)
    d0 = lax.rsqrt(deg0_ref[...] + 1.0)
    d1 = lax.rsqrt(deg1_ref[...] + 1.0)
    t1 = h[:, :DH] * d0
    t2 = h[:, DH:] * d1
    t1_ref[...] = t1
    t2_ref[...] = t2
    t1q_ref[...] = _pack_tbl(t1)
    t2q_ref[...] = _pack_tbl(t2)


_pre = pl.pallas_call(
    _pre_body,
    grid=(N // BR,),
    in_specs=[
        pl.BlockSpec((BR, DIN), lambda i: (i, 0)),
        pl.BlockSpec((DIN, DIN), lambda i: (0, 0)),
        pl.BlockSpec((BR, 1), lambda i: (i, 0)),
        pl.BlockSpec((BR, 1), lambda i: (i, 0)),
    ],
    out_specs=[pl.BlockSpec((BR, DH), lambda i: (i, 0))] * 2
              + [pl.BlockSpec((BR, DH // 2), lambda i: (i, 0))] * 2,
    out_shape=[jax.ShapeDtypeStruct((N_PAD, DH), jnp.float32)] * 2
             + [jax.ShapeDtypeStruct((N, DH // 2), jnp.uint32)] * 2,
)


def _fin_body(accs_ref, acch_ref, t1_ref, t2_ref, deg0_ref, deg1_ref,
              b1_ref, b2_ref, o_ref):
    d0 = lax.rsqrt(deg0_ref[...] + 1.0)
    d1 = lax.rsqrt(deg1_ref[...] + 1.0)
    left = (accs_ref[0] + t1_ref[...]) * d0 + b1_ref[...]
    t2p = (accs_ref[1] + t2_ref[...]) / (deg1_ref[...] + 1.0)
    right = (acch_ref[0] + acch_ref[1] + t2p) * d1 + b2_ref[...]
    o_ref[...] = jnp.concatenate([left, right], axis=1)


_fin = pl.pallas_call(
    _fin_body,
    grid=(N // BR,),
    in_specs=[
        pl.BlockSpec((2, BR, DH), lambda i: (0, i, 0)),
        pl.BlockSpec((2, BR, DH), lambda i: (0, i, 0)),
        pl.BlockSpec((BR, DH), lambda i: (i, 0)),
        pl.BlockSpec((BR, DH), lambda i: (i, 0)),
        pl.BlockSpec((BR, 1), lambda i: (i, 0)),
        pl.BlockSpec((BR, 1), lambda i: (i, 0)),
        pl.BlockSpec((1, DH), lambda i: (0, 0)),
        pl.BlockSpec((1, DH), lambda i: (0, 0)),
    ],
    out_specs=pl.BlockSpec((BR, DIN), lambda i: (i, 0)),
    out_shape=jax.ShapeDtypeStruct((N, DIN), jnp.float32),
)


def kernel(x, edges, W1, b1, W2, b2):
    pad = E_PAD - E
    zpad = jnp.zeros((pad,), jnp.int32)
    dpad = jnp.full((pad,), N, jnp.int32)
    srcs = jnp.stack([
        jnp.concatenate([edges[0, 0], zpad]),
        jnp.concatenate([edges[1, 0], zpad]),
    ]).reshape(2, NCH, CH)
    dsts = jnp.stack([
        jnp.concatenate([edges[0, 1], dpad]),
        jnp.concatenate([edges[1, 1], dpad]),
    ]).reshape(2, NCH, CH)

    degp = _hist(dsts)
    deg0 = degp[0, :N].reshape(N, 1)
    deg1 = degp[1, :N].reshape(N, 1)

    wcat = jnp.concatenate([W1, W2], axis=1)
    t1, t2, t1q, t2q = _pre(x, wcat, deg0, deg1)
    accs = _dual(t1q, t2q, srcs, dsts)
    acch, _tq = _hop2(accs, t2, degp, srcs, dsts)
    return _fin(accs, acch, t1, t2, deg0, deg1,
                b1.reshape(1, DH), b2.reshape(1, DH))
